# Initial kernel scaffold; baseline (speedup 1.0000x reference)
import functools

import jax
import jax.numpy as jnp
from jax import lax
from jax.experimental import pallas as pl
from jax.experimental.pallas import tpu as pltpu
from jax.experimental.pallas import tpu_sc as plsc

N = 100000
INC = 128
OUTC = 128
K = 27
EK = 23000
E = K * EK

NC = 2
NS = 16
NW = NC * NS
CHUNK = 512
CHUNKS_PER_W = 38
E_PAD = NW * CHUNK * CHUNKS_PER_W

BLK = 1000
NBLK = E // BLK


def _sc_gather(features, src_pad):
    mesh = plsc.VectorSubcoreMesh(core_axis_name="c", subcore_axis_name="s")

    @functools.partial(
        pl.kernel,
        out_type=jax.ShapeDtypeStruct((E_PAD, INC), jnp.float32),
        mesh=mesh,
        scratch_types=[
            pltpu.VMEM((CHUNK,), jnp.int32),
            pltpu.VMEM((CHUNK, INC), jnp.float32),
            pltpu.SemaphoreType.DMA,
        ],
    )
    def k(feat_hbm, src_hbm, out_hbm, idx_v, rows_v, sem):
        wid = lax.axis_index("s") * NC + lax.axis_index("c")

        @pl.loop(0, CHUNKS_PER_W)
        def _(j):
            base = (wid * CHUNKS_PER_W + j) * CHUNK
            pltpu.sync_copy(src_hbm.at[pl.ds(base, CHUNK)], idx_v)
            pltpu.async_copy(feat_hbm.at[idx_v], rows_v, sem).wait()
            pltpu.sync_copy(rows_v, out_hbm.at[pl.ds(base, CHUNK)])

    return k(features, src_pad)


def _tc_matmul_scatter(gathered, weights, dst2):
    def body(dst_ref, g_ref, w_ref, out_ref, t_ref):
        @pl.when(pl.program_id(0) == 0)
        def _():
            out_ref[...] = jnp.zeros_like(out_ref)

        t_ref[...] = jnp.dot(g_ref[...], w_ref[0],
                             preferred_element_type=jnp.float32)

        def add_row(i, _):
            d = dst_ref[0, i]
            out_ref[pl.ds(d, 1), :] += t_ref[pl.ds(i, 1), :]
            return 0

        lax.fori_loop(0, BLK, add_row, 0)

    return pl.pallas_call(
        body,
        grid=(NBLK,),
        in_specs=[
            pl.BlockSpec((1, BLK), lambda i: (i, 0), memory_space=pltpu.SMEM),
            pl.BlockSpec((BLK, INC), lambda i: (i, 0)),
            pl.BlockSpec((1, INC, OUTC), lambda i: (i // (EK // BLK), 0, 0)),
        ],
        out_specs=pl.BlockSpec((N, OUTC), lambda i: (0, 0)),
        out_shape=jax.ShapeDtypeStruct((N, OUTC), jnp.float32),
        scratch_shapes=[pltpu.VMEM((BLK, OUTC), jnp.float32)],
    )(dst2, gathered, weights)


def kernel(features, nbmap, coords, kernel):
    src = nbmap[:, :, 0].reshape(-1)
    src_pad = jnp.concatenate([src, jnp.zeros((E_PAD - E,), jnp.int32)])
    dst2 = nbmap[:, :, 1].reshape(NBLK, BLK)
    gathered = _sc_gather(features, src_pad)
    return _tc_matmul_scatter(gathered[:E], kernel, dst2)


# speedup vs baseline: 1.1047x; 1.1047x over previous
import functools

import jax
import jax.numpy as jnp
from jax import lax
from jax.experimental import pallas as pl
from jax.experimental.pallas import tpu as pltpu
from jax.experimental.pallas import tpu_sc as plsc

N = 100000
INC = 128
OUTC = 128
K = 27
EK = 23000
E = K * EK

NC = 2
NS = 16
NW = NC * NS
CHUNK = 512
CHUNKS_PER_W = 38
E_PAD = NW * CHUNK * CHUNKS_PER_W

BLK = 1000
NBLK = E // BLK


def _sc_gather(features, src_pad):
    mesh = plsc.VectorSubcoreMesh(core_axis_name="c", subcore_axis_name="s")

    @functools.partial(
        pl.kernel,
        out_type=jax.ShapeDtypeStruct((E_PAD, INC), jnp.float32),
        mesh=mesh,
        scratch_types=[
            pltpu.VMEM((CHUNK,), jnp.int32),
            pltpu.VMEM((CHUNK, INC), jnp.float32),
            pltpu.SemaphoreType.DMA,
        ],
    )
    def k(feat_hbm, src_hbm, out_hbm, idx_v, rows_v, sem):
        wid = lax.axis_index("s") * NC + lax.axis_index("c")

        @pl.loop(0, CHUNKS_PER_W)
        def _(j):
            base = (wid * CHUNKS_PER_W + j) * CHUNK
            pltpu.sync_copy(src_hbm.at[pl.ds(base, CHUNK)], idx_v)
            pltpu.async_copy(feat_hbm.at[idx_v], rows_v, sem).wait()
            pltpu.sync_copy(rows_v, out_hbm.at[pl.ds(base, CHUNK)])

    return k(features, src_pad)


def _tc_matmul_scatter(gathered, weights, dst2):
    def body(dst_ref, g_ref, w_ref, out_ref, t_ref):
        @pl.when(pl.program_id(0) == 0)
        def _():
            out_ref[...] = jnp.zeros_like(out_ref)

        t_ref[...] = jnp.dot(g_ref[...], w_ref[0],
                             preferred_element_type=jnp.float32)

        def add_row(i, _):
            d = dst_ref[0, 0, i]
            out_ref[pl.ds(d, 1), :] += t_ref[pl.ds(i, 1), :]
            return 0

        lax.fori_loop(0, BLK, add_row, 0)

    return pl.pallas_call(
        body,
        grid=(NBLK,),
        in_specs=[
            pl.BlockSpec((1, 1, BLK), lambda i: (i, 0, 0),
                         memory_space=pltpu.SMEM),
            pl.BlockSpec((BLK, INC), lambda i: (i, 0)),
            pl.BlockSpec((1, INC, OUTC), lambda i: (i // (EK // BLK), 0, 0)),
        ],
        out_specs=pl.BlockSpec((N, OUTC), lambda i: (0, 0)),
        out_shape=jax.ShapeDtypeStruct((N, OUTC), jnp.float32),
        scratch_shapes=[pltpu.VMEM((BLK, OUTC), jnp.float32)],
    )(dst2, gathered, weights)


def kernel(features, nbmap, coords, kernel):
    src = nbmap[:, :, 0].reshape(-1)
    src_pad = jnp.concatenate([src, jnp.zeros((E_PAD - E,), jnp.int32)])
    dst2 = nbmap[:, :, 1].reshape(NBLK, 1, BLK)
    gathered = _sc_gather(features, src_pad)
    return _tc_matmul_scatter(gathered, kernel, dst2)


# speedup vs baseline: 3.1494x; 2.8509x over previous
import dataclasses
import functools

import jax
import jax.numpy as jnp
from jax import lax
from jax.experimental import pallas as pl
from jax.experimental.pallas import tpu as pltpu
from jax.experimental.pallas import tpu_sc as plsc

N = 100000
INC = 128
OUTC = 128
K = 27
EK = 23000
E = K * EK

NC = 2
NS = 16
NW = NC * NS
CHUNK = 512
CHUNKS_PER_W = 38
E_PAD = NW * CHUNK * CHUNKS_PER_W

BLK = 1000
NBLK = E // BLK


def _sc_compiler_params():
    cp = pltpu.CompilerParams()
    if "needs_layout_passes" in pltpu.CompilerParams.__dataclass_fields__:
        cp = dataclasses.replace(cp, needs_layout_passes=False)
    return cp


def _sc_gather(features, src_pad):
    mesh = plsc.VectorSubcoreMesh(core_axis_name="c", subcore_axis_name="s")

    @functools.partial(
        pl.kernel,
        out_type=jax.ShapeDtypeStruct((E_PAD, INC), jnp.float32),
        mesh=mesh,
        scratch_types=[
            pltpu.VMEM((CHUNK,), jnp.int32),
            pltpu.VMEM((CHUNK, INC), jnp.float32),
            pltpu.SemaphoreType.DMA,
        ],
    )
    def k(feat_hbm, src_hbm, out_hbm, idx_v, rows_v, sem):
        wid = lax.axis_index("s") * NC + lax.axis_index("c")

        @pl.loop(0, CHUNKS_PER_W)
        def _(j):
            base = (wid * CHUNKS_PER_W + j) * CHUNK
            pltpu.sync_copy(src_hbm.at[pl.ds(base, CHUNK)], idx_v)
            pltpu.async_copy(feat_hbm.at[idx_v], rows_v, sem).wait()
            pltpu.sync_copy(rows_v, out_hbm.at[pl.ds(base, CHUNK)])

    return k(features, src_pad)


def _tc_matmul(gathered, weights):
    def body(g_ref, w_ref, t_ref):
        t_ref[...] = jnp.dot(g_ref[...], w_ref[0],
                             preferred_element_type=jnp.float32)

    return pl.pallas_call(
        body,
        grid=(NBLK,),
        in_specs=[
            pl.BlockSpec((BLK, INC), lambda i: (i, 0)),
            pl.BlockSpec((1, INC, OUTC), lambda i: (i // (EK // BLK), 0, 0)),
        ],
        out_specs=pl.BlockSpec((BLK, OUTC), lambda i: (i, 0)),
        out_shape=jax.ShapeDtypeStruct((E, OUTC), jnp.float32),
    )(gathered, weights)


NBKT = 25
BROWS = 4096
SHIFT = 12
NSLOT = 13
SLICE = E_PAD // NS
STRIP = 2048
NSTRIP = SLICE // STRIP
VPS = STRIP // 16
ARENA = SLICE + 256
TRASH = BROWS
ACC_ROWS = BROWS + NS * 16


def _sc_scan(dst_pad):
    mesh = plsc.VectorSubcoreMesh(core_axis_name="c", subcore_axis_name="s")

    @functools.partial(
        pl.kernel,
        out_type=(
            jax.ShapeDtypeStruct((NC, NS, ARENA), jnp.int32),
            jax.ShapeDtypeStruct((NC, NS, ARENA), jnp.int32),
            jax.ShapeDtypeStruct((NC, NS, 32), jnp.int32),
        ),
        mesh=mesh,
        compiler_params=_sc_compiler_params(),
        scratch_types=[
            pltpu.VMEM((STRIP,), jnp.int32),
            pltpu.VMEM((ARENA,), jnp.int32),
            pltpu.VMEM((ARENA,), jnp.int32),
            pltpu.VMEM((32,), jnp.int32),
        ],
    )
    def k(dst_hbm, e_out, loc_out, meta_out, dstb_v, e_ar, loc_ar, meta_v):
        c = lax.axis_index("c")
        t = lax.axis_index("s")
        iota = lax.iota(jnp.int32, 16)
        base = t * SLICE

        def strip1(s_i, cnts):
            pltpu.sync_copy(dst_hbm.at[pl.ds(base + s_i * STRIP, STRIP)],
                            dstb_v)

            def vec1(v, cnts):
                d = dstb_v[pl.ds(v * 16, 16)]
                bkt = d >> SHIFT
                return tuple(
                    cnts[s] + jnp.sum(jnp.where(bkt == 2 * s + c, 1, 0))
                    for s in range(NSLOT))

            return lax.fori_loop(0, VPS, vec1, cnts)

        cnts = lax.fori_loop(0, NSTRIP, strip1,
                             tuple(jnp.int32(0) for _ in range(NSLOT)))

        offs = []
        run = jnp.int32(0)
        meta_lo = jnp.zeros((16,), jnp.int32)
        for s in range(NSLOT):
            offs.append(run)
            meta_lo = jnp.where(iota == s, run, meta_lo)
            run = run + (((cnts[s] + 15) >> 4) << 4)

        def strip2(s_i, curs):
            pltpu.sync_copy(dst_hbm.at[pl.ds(base + s_i * STRIP, STRIP)],
                            dstb_v)

            def vec2(v, curs):
                d = dstb_v[pl.ds(v * 16, 16)]
                bkt = d >> SHIFT
                e_vec = base + s_i * STRIP + v * 16 + iota
                loc_vec = jnp.bitwise_and(d, BROWS - 1)
                new = []
                for s in range(NSLOT):
                    m = bkt == 2 * s + c
                    mi = jnp.where(m, 1, 0)
                    r = plsc.cumsum(mi)
                    pos = curs[s] + r - 1
                    plsc.store_scatter(e_ar, [pos], e_vec, mask=m)
                    plsc.store_scatter(loc_ar, [pos], loc_vec, mask=m)
                    new.append(curs[s] + jnp.sum(mi))
                return tuple(new)

            return lax.fori_loop(0, VPS, vec2, curs)

        curs = lax.fori_loop(0, NSTRIP, strip2, tuple(offs))

        trash = TRASH + t * 16 + iota
        meta_hi = jnp.zeros((16,), jnp.int32)
        for s in range(NSLOT):
            pad_n = jnp.bitwise_and(-(curs[s] - offs[s]), 15)
            pos = curs[s] + iota
            m = iota < pad_n
            plsc.store_scatter(e_ar, [pos], iota, mask=m)
            plsc.store_scatter(loc_ar, [pos], trash, mask=m)
            meta_hi = jnp.where(iota == s, curs[s] + pad_n, meta_hi)

        meta_v[pl.ds(0, 16)] = meta_lo
        meta_v[pl.ds(16, 16)] = meta_hi
        pltpu.sync_copy(e_ar, e_out.at[c, t])
        pltpu.sync_copy(loc_ar, loc_out.at[c, t])
        pltpu.sync_copy(meta_v, meta_out.at[c, t])

    return k(dst_pad)


def _sc_scatter_add(transformed, e_out, loc_out, meta_out):
    mesh = plsc.VectorSubcoreMesh(core_axis_name="c", subcore_axis_name="s")

    @functools.partial(
        pl.kernel,
        out_type=jax.ShapeDtypeStruct((N, OUTC), jnp.float32),
        mesh=mesh,
        compiler_params=_sc_compiler_params(),
        scratch_types=[
            pltpu.VMEM((ARENA,), jnp.int32),
            pltpu.VMEM((ARENA,), jnp.int32),
            pltpu.VMEM((32,), jnp.int32),
            pltpu.VMEM((16, OUTC), jnp.float32),
            pltpu.VMEM((16, OUTC), jnp.float32),
            pltpu.VMEM((16, OUTC), jnp.float32),
            pltpu.VMEM_SHARED((ACC_ROWS, OUTC), jnp.float32),
            pltpu.SemaphoreType.DMA,
            pltpu.SemaphoreType.DMA,
        ],
    )
    def k(t_hbm, e_hbm, loc_hbm, meta_hbm, out_hbm,
          e_ar, loc_ar, meta_v, zero_v, buf0, buf1, acc, sem0, sem1):
        c = lax.axis_index("c")
        t = lax.axis_index("s")
        iota = lax.iota(jnp.int32, 16)
        slab = BROWS // NS

        pltpu.sync_copy(e_hbm.at[c, t], e_ar)
        pltpu.sync_copy(loc_hbm.at[c, t], loc_ar)
        pltpu.sync_copy(meta_hbm.at[c, t], meta_v)
        for r in range(16):
            for q in range(OUTC // 16):
                zero_v[r, pl.ds(q * 16, 16)] = jnp.zeros((16,), jnp.float32)
        meta_lo = meta_v[pl.ds(0, 16)]
        meta_hi = meta_v[pl.ds(16, 16)]

        def accumulate(s):
            @pl.loop(0, slab // 16)
            def _(i):
                pltpu.sync_copy(zero_v, acc.at[pl.ds(t * slab + i * 16, 16)])

            plsc.subcore_barrier()
            g0 = jnp.sum(jnp.where(iota == s, meta_lo, 0)) >> 4
            g1 = jnp.sum(jnp.where(iota == s, meta_hi, 0)) >> 4

            def granule(g, buf, sem):
                e16 = e_ar[pl.ds(g * 16, 16)]
                return pltpu.async_copy(t_hbm.at[e16], buf, sem)

            def add(g, buf):
                l16 = loc_ar[pl.ds(g * 16, 16)]
                pltpu.sync_copy(buf, acc.at[l16], add=True)

            def pair_body(i, _):
                g = g0 + i * 2
                cp0 = granule(g, buf0, sem0)
                has1 = g + 1 < g1

                @pl.when(has1)
                def _():
                    granule(g + 1, buf1, sem1)

                cp0.wait()
                add(g, buf0)

                @pl.when(has1)
                def _():
                    pltpu.make_async_copy(
                        t_hbm.at[e_ar[pl.ds((g + 1) * 16, 16)]],
                        buf1, sem1).wait()
                    add(g + 1, buf1)

                return 0

            lax.fori_loop(0, (g1 - g0 + 1) >> 1, pair_body, 0)
            plsc.subcore_barrier()

        def bucket_body(s, _):
            accumulate(s)
            b = 2 * s + c
            pltpu.sync_copy(
                acc.at[pl.ds(t * slab, slab)],
                out_hbm.at[pl.ds(b * BROWS + t * slab, slab)])
            return 0

        lax.fori_loop(0, (NBKT - 1) // 2, bucket_body, 0)

        tail_rows = N - (NBKT - 1) * BROWS
        full = tail_rows // slab
        rem = tail_rows % slab

        @pl.when(c == (NBKT - 1) % 2)
        def _():
            accumulate(jnp.int32((NBKT - 1) // 2))

            @pl.when(t < full)
            def _():
                pltpu.sync_copy(
                    acc.at[pl.ds(t * slab, slab)],
                    out_hbm.at[pl.ds((NBKT - 1) * BROWS + t * slab, slab)])

            if rem:

                @pl.when(t == full)
                def _():
                    pltpu.sync_copy(
                        acc.at[pl.ds(full * slab, rem)],
                        out_hbm.at[
                            pl.ds((NBKT - 1) * BROWS + full * slab, rem)])

    return k(transformed, e_out, loc_out, meta_out)


def kernel(features, nbmap, coords, kernel):
    src = nbmap[:, :, 0].reshape(-1)
    src_pad = jnp.concatenate([src, jnp.zeros((E_PAD - E,), jnp.int32)])
    dst = nbmap[:, :, 1].reshape(-1)
    dst_pad = jnp.concatenate(
        [dst, jnp.full((E_PAD - E,), 1 << 20, jnp.int32)])
    gathered = _sc_gather(features, src_pad)
    transformed = _tc_matmul(gathered, kernel)
    e_out, loc_out, meta_out = _sc_scan(dst_pad)
    return _sc_scatter_add(transformed, e_out, loc_out, meta_out)


# speedup vs baseline: 3.1521x; 1.0009x over previous
import dataclasses
import functools

import jax
import jax.numpy as jnp
from jax import lax
from jax.experimental import pallas as pl
from jax.experimental.pallas import tpu as pltpu
from jax.experimental.pallas import tpu_sc as plsc

N = 100000
INC = 128
OUTC = 128
K = 27
EK = 23000
E = K * EK

NC = 2
NS = 16
NW = NC * NS
CHUNK = 512
CHUNKS_PER_W = 38
E_PAD = NW * CHUNK * CHUNKS_PER_W

BLK = 1000
NBLK = E // BLK


def _sc_compiler_params():
    cp = pltpu.CompilerParams()
    if "needs_layout_passes" in pltpu.CompilerParams.__dataclass_fields__:
        cp = dataclasses.replace(cp, needs_layout_passes=False)
    return cp


def _sc_gather(features, src_pad):
    mesh = plsc.VectorSubcoreMesh(core_axis_name="c", subcore_axis_name="s")

    @functools.partial(
        pl.kernel,
        out_type=jax.ShapeDtypeStruct((E_PAD, INC), jnp.float32),
        mesh=mesh,
        scratch_types=[
            pltpu.VMEM((CHUNK,), jnp.int32),
            pltpu.VMEM((CHUNK, INC), jnp.float32),
            pltpu.SemaphoreType.DMA,
        ],
    )
    def k(feat_hbm, src_hbm, out_hbm, idx_v, rows_v, sem):
        wid = lax.axis_index("s") * NC + lax.axis_index("c")

        @pl.loop(0, CHUNKS_PER_W)
        def _(j):
            base = (wid * CHUNKS_PER_W + j) * CHUNK
            pltpu.sync_copy(src_hbm.at[pl.ds(base, CHUNK)], idx_v)
            pltpu.async_copy(feat_hbm.at[idx_v], rows_v, sem).wait()
            pltpu.sync_copy(rows_v, out_hbm.at[pl.ds(base, CHUNK)])

    return k(features, src_pad)


def _tc_matmul(gathered, weights):
    def body(g_ref, w_ref, t_ref):
        t_ref[...] = jnp.dot(g_ref[...].astype(jnp.bfloat16),
                             w_ref[0].astype(jnp.bfloat16),
                             preferred_element_type=jnp.float32)

    return pl.pallas_call(
        body,
        grid=(NBLK,),
        in_specs=[
            pl.BlockSpec((BLK, INC), lambda i: (i, 0)),
            pl.BlockSpec((1, INC, OUTC), lambda i: (i // (EK // BLK), 0, 0)),
        ],
        out_specs=pl.BlockSpec((BLK, OUTC), lambda i: (i, 0)),
        out_shape=jax.ShapeDtypeStruct((E, OUTC), jnp.float32),
    )(gathered, weights)


NBKT = 25
BROWS = 4096
SHIFT = 12
NSLOT = 13
SLICE = E_PAD // NS
STRIP = 2048
NSTRIP = SLICE // STRIP
VPS = STRIP // 16
ARENA = SLICE + 256
TRASH = BROWS
ACC_ROWS = BROWS + NS * 16


def _sc_scan(dst_pad):
    mesh = plsc.VectorSubcoreMesh(core_axis_name="c", subcore_axis_name="s")

    @functools.partial(
        pl.kernel,
        out_type=(
            jax.ShapeDtypeStruct((NC, NS, ARENA), jnp.int32),
            jax.ShapeDtypeStruct((NC, NS, ARENA), jnp.int32),
            jax.ShapeDtypeStruct((NC, NS, 32), jnp.int32),
        ),
        mesh=mesh,
        compiler_params=_sc_compiler_params(),
        scratch_types=[
            pltpu.VMEM((STRIP,), jnp.int32),
            pltpu.VMEM((ARENA,), jnp.int32),
            pltpu.VMEM((ARENA,), jnp.int32),
            pltpu.VMEM((32,), jnp.int32),
        ],
    )
    def k(dst_hbm, e_out, loc_out, meta_out, dstb_v, e_ar, loc_ar, meta_v):
        c = lax.axis_index("c")
        t = lax.axis_index("s")
        iota = lax.iota(jnp.int32, 16)
        base = t * SLICE

        def strip1(s_i, cnts):
            pltpu.sync_copy(dst_hbm.at[pl.ds(base + s_i * STRIP, STRIP)],
                            dstb_v)

            def vec1(v, cnts):
                d = dstb_v[pl.ds(v * 16, 16)]
                bkt = d >> SHIFT
                return tuple(
                    cnts[s] + jnp.sum(jnp.where(bkt == 2 * s + c, 1, 0))
                    for s in range(NSLOT))

            return lax.fori_loop(0, VPS, vec1, cnts)

        cnts = lax.fori_loop(0, NSTRIP, strip1,
                             tuple(jnp.int32(0) for _ in range(NSLOT)))

        offs = []
        run = jnp.int32(0)
        meta_lo = jnp.zeros((16,), jnp.int32)
        for s in range(NSLOT):
            offs.append(run)
            meta_lo = jnp.where(iota == s, run, meta_lo)
            run = run + (((cnts[s] + 15) >> 4) << 4)

        def strip2(s_i, curs):
            pltpu.sync_copy(dst_hbm.at[pl.ds(base + s_i * STRIP, STRIP)],
                            dstb_v)

            def vec2(v, curs):
                d = dstb_v[pl.ds(v * 16, 16)]
                bkt = d >> SHIFT
                e_vec = base + s_i * STRIP + v * 16 + iota
                loc_vec = jnp.bitwise_and(d, BROWS - 1)
                new = []
                for s in range(NSLOT):
                    m = bkt == 2 * s + c
                    mi = jnp.where(m, 1, 0)
                    r = plsc.cumsum(mi)
                    pos = curs[s] + r - 1
                    plsc.store_scatter(e_ar, [pos], e_vec, mask=m)
                    plsc.store_scatter(loc_ar, [pos], loc_vec, mask=m)
                    new.append(curs[s] + jnp.sum(mi))
                return tuple(new)

            return lax.fori_loop(0, VPS, vec2, curs)

        curs = lax.fori_loop(0, NSTRIP, strip2, tuple(offs))

        trash = TRASH + t * 16 + iota
        meta_hi = jnp.zeros((16,), jnp.int32)
        for s in range(NSLOT):
            pad_n = jnp.bitwise_and(-(curs[s] - offs[s]), 15)
            pos = curs[s] + iota
            m = iota < pad_n
            plsc.store_scatter(e_ar, [pos], iota, mask=m)
            plsc.store_scatter(loc_ar, [pos], trash, mask=m)
            meta_hi = jnp.where(iota == s, curs[s] + pad_n, meta_hi)

        meta_v[pl.ds(0, 16)] = meta_lo
        meta_v[pl.ds(16, 16)] = meta_hi
        pltpu.sync_copy(e_ar, e_out.at[c, t])
        pltpu.sync_copy(loc_ar, loc_out.at[c, t])
        pltpu.sync_copy(meta_v, meta_out.at[c, t])

    return k(dst_pad)


def _sc_scatter_add(transformed, e_out, loc_out, meta_out):
    mesh = plsc.VectorSubcoreMesh(core_axis_name="c", subcore_axis_name="s")

    @functools.partial(
        pl.kernel,
        out_type=jax.ShapeDtypeStruct((N, OUTC), jnp.float32),
        mesh=mesh,
        compiler_params=_sc_compiler_params(),
        scratch_types=[
            pltpu.VMEM((ARENA,), jnp.int32),
            pltpu.VMEM((ARENA,), jnp.int32),
            pltpu.VMEM((32,), jnp.int32),
            pltpu.VMEM((16, OUTC), jnp.float32),
            pltpu.VMEM((16, OUTC), jnp.float32),
            pltpu.VMEM((16, OUTC), jnp.float32),
            pltpu.VMEM_SHARED((ACC_ROWS, OUTC), jnp.float32),
            pltpu.SemaphoreType.DMA,
            pltpu.SemaphoreType.DMA,
        ],
    )
    def k(t_hbm, e_hbm, loc_hbm, meta_hbm, out_hbm,
          e_ar, loc_ar, meta_v, zero_v, buf0, buf1, acc, sem0, sem1):
        c = lax.axis_index("c")
        t = lax.axis_index("s")
        iota = lax.iota(jnp.int32, 16)
        slab = BROWS // NS

        pltpu.sync_copy(e_hbm.at[c, t], e_ar)
        pltpu.sync_copy(loc_hbm.at[c, t], loc_ar)
        pltpu.sync_copy(meta_hbm.at[c, t], meta_v)
        for r in range(16):
            for q in range(OUTC // 16):
                zero_v[r, pl.ds(q * 16, 16)] = jnp.zeros((16,), jnp.float32)
        meta_lo = meta_v[pl.ds(0, 16)]
        meta_hi = meta_v[pl.ds(16, 16)]

        def accumulate(s):
            @pl.loop(0, slab // 16)
            def _(i):
                pltpu.sync_copy(zero_v, acc.at[pl.ds(t * slab + i * 16, 16)])

            plsc.subcore_barrier()
            g0 = jnp.sum(jnp.where(iota == s, meta_lo, 0)) >> 4
            g1 = jnp.sum(jnp.where(iota == s, meta_hi, 0)) >> 4

            def granule(g, buf, sem):
                e16 = e_ar[pl.ds(g * 16, 16)]
                return pltpu.async_copy(t_hbm.at[e16], buf, sem)

            def add(g, buf):
                l16 = loc_ar[pl.ds(g * 16, 16)]
                pltpu.sync_copy(buf, acc.at[l16], add=True)

            def pair_body(i, _):
                g = g0 + i * 2
                cp0 = granule(g, buf0, sem0)
                has1 = g + 1 < g1

                @pl.when(has1)
                def _():
                    granule(g + 1, buf1, sem1)

                cp0.wait()
                add(g, buf0)

                @pl.when(has1)
                def _():
                    pltpu.make_async_copy(
                        t_hbm.at[e_ar[pl.ds((g + 1) * 16, 16)]],
                        buf1, sem1).wait()
                    add(g + 1, buf1)

                return 0

            lax.fori_loop(0, (g1 - g0 + 1) >> 1, pair_body, 0)
            plsc.subcore_barrier()

        def bucket_body(s, _):
            accumulate(s)
            b = 2 * s + c
            pltpu.sync_copy(
                acc.at[pl.ds(t * slab, slab)],
                out_hbm.at[pl.ds(b * BROWS + t * slab, slab)])
            return 0

        lax.fori_loop(0, (NBKT - 1) // 2, bucket_body, 0)

        tail_rows = N - (NBKT - 1) * BROWS
        full = tail_rows // slab
        rem = tail_rows % slab

        @pl.when(c == (NBKT - 1) % 2)
        def _():
            accumulate(jnp.int32((NBKT - 1) // 2))

            @pl.when(t < full)
            def _():
                pltpu.sync_copy(
                    acc.at[pl.ds(t * slab, slab)],
                    out_hbm.at[pl.ds((NBKT - 1) * BROWS + t * slab, slab)])

            if rem:

                @pl.when(t == full)
                def _():
                    pltpu.sync_copy(
                        acc.at[pl.ds(full * slab, rem)],
                        out_hbm.at[
                            pl.ds((NBKT - 1) * BROWS + full * slab, rem)])

    return k(transformed, e_out, loc_out, meta_out)


def kernel(features, nbmap, coords, kernel):
    src = nbmap[:, :, 0].reshape(-1)
    src_pad = jnp.concatenate([src, jnp.zeros((E_PAD - E,), jnp.int32)])
    dst = nbmap[:, :, 1].reshape(-1)
    dst_pad = jnp.concatenate(
        [dst, jnp.full((E_PAD - E,), 1 << 20, jnp.int32)])
    e_out, loc_out, meta_out = _sc_scan(dst_pad)
    gathered = _sc_gather(features, src_pad)
    transformed = _tc_matmul(gathered, kernel)
    return _sc_scatter_add(transformed, e_out, loc_out, meta_out)
